# Initial kernel scaffold; baseline (speedup 1.0000x reference)
import jax
import jax.numpy as jnp
from jax.experimental import pallas as pl


def kernel(af, pi, pt, si, phoneme_table, speaker_table, W1, b1, gamma, beta):
    raise NotImplementedError("write your pallas kernel here")


# speedup vs baseline: 1.7792x; 1.7792x over previous
import functools

import jax
import jax.numpy as jnp
from jax import lax
from jax.experimental import pallas as pl
from jax.experimental.pallas import tpu as pltpu
from jax.experimental.pallas import tpu_sc as plsc

FEAT_DIM = 80
D_MODEL = 256
PH_DIM = 32
SP_DIM = 8
E_DIM = 48
N_TOK = 32 * 1500

_NC = 2
_NS = 16
_NW = _NC * _NS
_CHUNK = 128
_B_PER_W = 1536
_NPAD = _NW * _B_PER_W

_R = 2400


def _sc_gather(pi_pad, si_pad, table):
    mesh = plsc.VectorSubcoreMesh(core_axis_name="c", subcore_axis_name="s")

    @functools.partial(
        pl.kernel,
        mesh=mesh,
        out_type=jax.ShapeDtypeStruct((_NPAD, E_DIM), jnp.float32),
        compiler_params=pltpu.CompilerParams(use_tc_tiling_on_sc=False),
        scratch_types=[
            pltpu.VMEM((_B_PER_W,), jnp.int32),
            pltpu.VMEM((_B_PER_W,), jnp.int32),
            pltpu.VMEM((_B_PER_W,), jnp.int32),
            pltpu.VMEM((_B_PER_W, E_DIM), jnp.float32),
            pltpu.SemaphoreType.DMA,
        ],
    )
    def gather_kernel(pi_hbm, si_hbm, tab_hbm, out_hbm, pi_v, si_v, idx_v, rows_v, sem):
        wid = lax.axis_index("s") * _NC + lax.axis_index("c")
        base = wid * _B_PER_W
        pltpu.sync_copy(pi_hbm.at[pl.ds(base, _B_PER_W)], pi_v)
        pltpu.sync_copy(si_hbm.at[pl.ds(base, _B_PER_W)], si_v)

        def combine(j, carry):
            s = pl.ds(j * 16, 16)
            idx_v[s] = pi_v[s] * 2 + si_v[s]
            return carry

        lax.fori_loop(0, _B_PER_W // 16, combine, 0)

        copies = [
            pltpu.async_copy(
                tab_hbm.at[idx_v.at[pl.ds(k * _CHUNK, _CHUNK)]],
                rows_v.at[pl.ds(k * _CHUNK, _CHUNK)],
                sem,
            )
            for k in range(_B_PER_W // _CHUNK)
        ]
        for c in copies:
            c.wait()
        pltpu.sync_copy(rows_v, out_hbm.at[pl.ds(base, _B_PER_W)])

    return gather_kernel(pi_pad, si_pad, table)


def _tc_body(af_ref, pt_ref, e_ref, wa_ref, wpt_ref, we_ref, b1_ref, g_ref, bt_ref, out_ref):
    h = jnp.dot(af_ref[...], wa_ref[...], preferred_element_type=jnp.float32)
    h = h + jnp.dot(e_ref[...], we_ref[...], preferred_element_type=jnp.float32)
    h = h + pt_ref[...] * wpt_ref[...]
    h = h + b1_ref[...]
    mu = jnp.mean(h, axis=1, keepdims=True)
    var = jnp.mean((h - mu) ** 2, axis=1, keepdims=True)
    hn = (h - mu) * lax.rsqrt(var + 1e-5)
    hn = hn * g_ref[...] + bt_ref[...]
    out_ref[...] = 0.5 * hn * (1.0 + lax.erf(hn * 0.7071067811865476))


def kernel(af, pi, pt, si, phoneme_table, speaker_table, W1, b1, gamma, beta):
    af2 = af.reshape(N_TOK, FEAT_DIM)
    pt2 = pt.reshape(N_TOK, 1)
    pad = _NPAD - N_TOK
    pi_pad = jnp.concatenate([pi.reshape(-1).astype(jnp.int32),
                              jnp.zeros((pad,), jnp.int32)])
    si_pad = jnp.concatenate([si.reshape(-1).astype(jnp.int32),
                              jnp.zeros((pad,), jnp.int32)])

    comb = jnp.concatenate(
        [
            jnp.repeat(phoneme_table, 2, axis=0),
            jnp.tile(speaker_table, (128, 1)),
            jnp.zeros((256, E_DIM - PH_DIM - SP_DIM), jnp.float32),
        ],
        axis=1,
    )

    e = _sc_gather(pi_pad, si_pad, comb)

    wa = W1[:FEAT_DIM]
    wpt = W1[FEAT_DIM:FEAT_DIM + 1]
    we = jnp.concatenate(
        [W1[FEAT_DIM + 1:], jnp.zeros((E_DIM - PH_DIM - SP_DIM, D_MODEL), jnp.float32)]
    )

    grid = (N_TOK // _R,)
    out = pl.pallas_call(
        _tc_body,
        grid=grid,
        in_specs=[
            pl.BlockSpec((_R, FEAT_DIM), lambda i: (i, 0)),
            pl.BlockSpec((_R, 1), lambda i: (i, 0)),
            pl.BlockSpec((_R, E_DIM), lambda i: (i, 0)),
            pl.BlockSpec((FEAT_DIM, D_MODEL), lambda i: (0, 0)),
            pl.BlockSpec((1, D_MODEL), lambda i: (0, 0)),
            pl.BlockSpec((E_DIM, D_MODEL), lambda i: (0, 0)),
            pl.BlockSpec((1, D_MODEL), lambda i: (0, 0)),
            pl.BlockSpec((1, D_MODEL), lambda i: (0, 0)),
            pl.BlockSpec((1, D_MODEL), lambda i: (0, 0)),
        ],
        out_specs=pl.BlockSpec((_R, D_MODEL), lambda i: (i, 0)),
        out_shape=jax.ShapeDtypeStruct((N_TOK, D_MODEL), jnp.float32),
    )(af2, pt2, e, wa, wpt, we, b1.reshape(1, D_MODEL),
      gamma.reshape(1, D_MODEL), beta.reshape(1, D_MODEL))

    return out.reshape(32, 1500, D_MODEL)


# speedup vs baseline: 2.4499x; 1.3770x over previous
import functools

import jax
import jax.numpy as jnp
from jax import lax
from jax.experimental import pallas as pl
from jax.experimental.pallas import tpu as pltpu
from jax.experimental.pallas import tpu_sc as plsc

FEAT_DIM = 80
D_MODEL = 256
PH_DIM = 32
SP_DIM = 8
E_DIM = 128
B = 32
T = 1500
T_PAD = 1504

_NC = 2
_NS = 16
_NW = _NC * _NS

_ROUNDS = ((0, 384), (384, 384), (768, 384), (1152, 348))
_RMAX = 384


def _sc_gather(idx_flat, table):
    mesh = plsc.VectorSubcoreMesh(core_axis_name="c", subcore_axis_name="s")

    @functools.partial(
        pl.kernel,
        mesh=mesh,
        out_type=jax.ShapeDtypeStruct((B, T, E_DIM), jnp.float32),
        scratch_types=[
            pltpu.VMEM((T_PAD,), jnp.int32),
            pltpu.VMEM((2, _RMAX, E_DIM), jnp.float32),
            pltpu.SemaphoreType.DMA,
            pltpu.SemaphoreType.DMA,
        ],
    )
    def gather_kernel(idx_hbm, tab_hbm, out_hbm, idx_v, rows_v, gsem, wsem):
        b = lax.axis_index("s") * _NC + lax.axis_index("c")
        pltpu.sync_copy(idx_hbm.at[pl.ds(b * T_PAD, T_PAD)], idx_v)
        writes = []
        for r, (start, n) in enumerate(_ROUNDS):
            buf = rows_v.at[r % 2]
            gathers = []
            off = 0
            while off < n:
                c = min(128, n - off)
                gathers.append(
                    pltpu.async_copy(
                        tab_hbm.at[idx_v.at[pl.ds(start + off, c)]],
                        buf.at[pl.ds(off, c)],
                        gsem,
                    )
                )
                off += c
            if r >= 2:
                writes[r - 2].wait()
            for g in gathers:
                g.wait()
            writes.append(
                pltpu.async_copy(
                    buf.at[pl.ds(0, n)], out_hbm.at[b, pl.ds(start, n)], wsem
                )
            )
        for w in writes[-2:]:
            w.wait()

    return gather_kernel(idx_flat, table)


def _tc_body(af_ref, pt_ref, e_ref, wa_ref, wpt_ref, we_ref, b1_ref, g_ref, bt_ref, out_ref):
    h = jnp.dot(af_ref[0], wa_ref[...], preferred_element_type=jnp.float32)
    h = h + jnp.dot(e_ref[0], we_ref[...], preferred_element_type=jnp.float32)
    h = h + pt_ref[0] * wpt_ref[...]
    h = h + b1_ref[...]
    mu = jnp.mean(h, axis=1, keepdims=True)
    var = jnp.mean((h - mu) ** 2, axis=1, keepdims=True)
    hn = (h - mu) * lax.rsqrt(var + 1e-5)
    hn = hn * g_ref[...] + bt_ref[...]
    out_ref[0] = 0.5 * hn * (1.0 + lax.erf(hn * 0.7071067811865476))


def kernel(af, pi, pt, si, phoneme_table, speaker_table, W1, b1, gamma, beta):
    idx = pi.astype(jnp.int32) * 2 + si.astype(jnp.int32)
    idx_flat = jnp.pad(idx, ((0, 0), (0, T_PAD - T))).reshape(-1)

    comb = jnp.concatenate(
        [
            jnp.repeat(phoneme_table, 2, axis=0),
            jnp.tile(speaker_table, (128, 1)),
            jnp.zeros((256, E_DIM - PH_DIM - SP_DIM), jnp.float32),
        ],
        axis=1,
    )

    e = _sc_gather(idx_flat, comb)

    wa = W1[:FEAT_DIM]
    wpt = W1[FEAT_DIM:FEAT_DIM + 1]
    we = jnp.concatenate(
        [W1[FEAT_DIM + 1:], jnp.zeros((E_DIM - PH_DIM - SP_DIM, D_MODEL), jnp.float32)]
    )

    out = pl.pallas_call(
        _tc_body,
        grid=(B,),
        in_specs=[
            pl.BlockSpec((1, T, FEAT_DIM), lambda i: (i, 0, 0)),
            pl.BlockSpec((1, T, 1), lambda i: (i, 0, 0)),
            pl.BlockSpec((1, T, E_DIM), lambda i: (i, 0, 0)),
            pl.BlockSpec((FEAT_DIM, D_MODEL), lambda i: (0, 0)),
            pl.BlockSpec((1, D_MODEL), lambda i: (0, 0)),
            pl.BlockSpec((E_DIM, D_MODEL), lambda i: (0, 0)),
            pl.BlockSpec((1, D_MODEL), lambda i: (0, 0)),
            pl.BlockSpec((1, D_MODEL), lambda i: (0, 0)),
            pl.BlockSpec((1, D_MODEL), lambda i: (0, 0)),
        ],
        out_specs=pl.BlockSpec((1, T, D_MODEL), lambda i: (i, 0, 0)),
        out_shape=jax.ShapeDtypeStruct((B, T, D_MODEL), jnp.float32),
    )(af, pt, e, wa, wpt, we, b1.reshape(1, D_MODEL),
      gamma.reshape(1, D_MODEL), beta.reshape(1, D_MODEL))

    return out


# speedup vs baseline: 2.8204x; 1.1512x over previous
import functools

import jax
import jax.numpy as jnp
from jax import lax
from jax.experimental import pallas as pl
from jax.experimental.pallas import tpu as pltpu
from jax.experimental.pallas import tpu_sc as plsc

FEAT_DIM = 80
D_MODEL = 256
PH_DIM = 32
SP_DIM = 8
E_DIM = 128
B = 32
T = 1500
T_PAD = 1504

_NC = 2
_NS = 16
_NW = _NC * _NS

_ROUNDS = ((0, 384), (384, 384), (768, 384), (1152, 348))
_RMAX = 384


def _sc_gather(idx_flat, table):
    mesh = plsc.VectorSubcoreMesh(core_axis_name="c", subcore_axis_name="s")

    @functools.partial(
        pl.kernel,
        mesh=mesh,
        out_type=jax.ShapeDtypeStruct((B, T, E_DIM), jnp.float32),
        scratch_types=[
            pltpu.VMEM((T_PAD,), jnp.int32),
            pltpu.VMEM((2, _RMAX, E_DIM), jnp.float32),
            pltpu.SemaphoreType.DMA,
            pltpu.SemaphoreType.DMA,
        ],
    )
    def gather_kernel(idx_hbm, tab_hbm, out_hbm, idx_v, rows_v, gsem, wsem):
        b = lax.axis_index("s") * _NC + lax.axis_index("c")
        pltpu.sync_copy(idx_hbm.at[pl.ds(b * T_PAD, T_PAD)], idx_v)
        writes = []
        for r, (start, n) in enumerate(_ROUNDS):
            buf = rows_v.at[r % 2]
            gathers = []
            off = 0
            while off < n:
                c = min(128, n - off)
                gathers.append(
                    pltpu.async_copy(
                        tab_hbm.at[idx_v.at[pl.ds(start + off, c)]],
                        buf.at[pl.ds(off, c)],
                        gsem,
                    )
                )
                off += c
            if r >= 2:
                writes[r - 2].wait()
            for g in gathers:
                g.wait()
            writes.append(
                pltpu.async_copy(
                    buf.at[pl.ds(0, n)], out_hbm.at[b, pl.ds(start, n)], wsem
                )
            )
        for w in writes[-2:]:
            w.wait()

    return gather_kernel(idx_flat, table)


def _tc_body(aft_ref, ptt_ref, e_ref, wa_ref, wpt_ref, we_ref, b1_ref, g_ref, bt_ref, out_ref):
    cdims = (((0,), (0,)), ((), ()))
    h = lax.dot_general(aft_ref[0], wa_ref[...], cdims,
                        preferred_element_type=jnp.float32)
    h = h + jnp.dot(e_ref[0], we_ref[...], preferred_element_type=jnp.float32)
    h = h + lax.dot_general(ptt_ref[0], wpt_ref[...], cdims,
                            preferred_element_type=jnp.float32)
    h = h + b1_ref[...]
    mu = jnp.mean(h, axis=1, keepdims=True)
    var = jnp.mean((h - mu) ** 2, axis=1, keepdims=True)
    hn = (h - mu) * lax.rsqrt(var + 1e-5)
    hn = hn * g_ref[...] + bt_ref[...]
    out_ref[0] = 0.5 * hn * (1.0 + lax.erf(hn * 0.7071067811865476))


def kernel(af, pi, pt, si, phoneme_table, speaker_table, W1, b1, gamma, beta):
    idx = pi.astype(jnp.int32) * 2 + si.astype(jnp.int32)
    idx_flat = jnp.pad(idx, ((0, 0), (0, T_PAD - T))).reshape(-1)

    comb = jnp.concatenate(
        [
            jnp.repeat(phoneme_table, 2, axis=0),
            jnp.tile(speaker_table, (128, 1)),
            jnp.zeros((256, E_DIM - PH_DIM - SP_DIM), jnp.float32),
        ],
        axis=1,
    )

    e = _sc_gather(idx_flat, comb)

    wa = W1[:FEAT_DIM]
    wpt = W1[FEAT_DIM:FEAT_DIM + 1]
    we = jnp.concatenate(
        [W1[FEAT_DIM + 1:], jnp.zeros((E_DIM - PH_DIM - SP_DIM, D_MODEL), jnp.float32)]
    )

    out_l = pl.pallas_call(
        _tc_body,
        grid=(B,),
        in_specs=[
            pl.BlockSpec((1, FEAT_DIM, T), lambda i: (i, 0, 0)),
            pl.BlockSpec((1, 1, T), lambda i: (i, 0, 0)),
            pl.BlockSpec((1, T, E_DIM), lambda i: (i, 0, 0)),
            pl.BlockSpec((FEAT_DIM, D_MODEL), lambda i: (0, 0)),
            pl.BlockSpec((1, D_MODEL), lambda i: (0, 0)),
            pl.BlockSpec((E_DIM, D_MODEL), lambda i: (0, 0)),
            pl.BlockSpec((1, D_MODEL), lambda i: (0, 0)),
            pl.BlockSpec((1, D_MODEL), lambda i: (0, 0)),
            pl.BlockSpec((1, D_MODEL), lambda i: (0, 0)),
        ],
        out_specs=pl.BlockSpec((1, T, D_MODEL), lambda i: (i, 0, 0)),
        out_shape=jax.ShapeDtypeStruct((B, T, D_MODEL), jnp.float32),
    )(af.transpose(0, 2, 1), pt.transpose(0, 2, 1), e, wa, wpt, we,
      b1.reshape(1, D_MODEL), gamma.reshape(1, D_MODEL), beta.reshape(1, D_MODEL))

    return out_l


# speedup vs baseline: 3.3648x; 1.1930x over previous
import functools

import jax
import jax.numpy as jnp
from jax import lax
from jax.experimental import pallas as pl
from jax.experimental.pallas import tpu as pltpu
from jax.experimental.pallas import tpu_sc as plsc

FEAT_DIM = 80
D_MODEL = 256
PH_DIM = 32
SP_DIM = 8
E_DIM = 128
B = 32
T = 1500
T_PAD = 1504

_NC = 2
_NS = 16
_NW = _NC * _NS

_ROUNDS = ((0, 384), (384, 384), (768, 384), (1152, 348))
_RMAX = 384


def _sc_gather(idx_flat, table):
    mesh = plsc.VectorSubcoreMesh(core_axis_name="c", subcore_axis_name="s")

    @functools.partial(
        pl.kernel,
        mesh=mesh,
        out_type=jax.ShapeDtypeStruct((B, T, E_DIM), jnp.float32),
        scratch_types=[
            pltpu.VMEM((T_PAD,), jnp.int32),
            pltpu.VMEM((2, _RMAX, E_DIM), jnp.float32),
            pltpu.SemaphoreType.DMA,
            pltpu.SemaphoreType.DMA,
        ],
    )
    def gather_kernel(idx_hbm, tab_hbm, out_hbm, idx_v, rows_v, gsem, wsem):
        b = lax.axis_index("s") * _NC + lax.axis_index("c")
        pltpu.sync_copy(idx_hbm.at[pl.ds(b * T_PAD, T_PAD)], idx_v)
        writes = []
        for r, (start, n) in enumerate(_ROUNDS):
            buf = rows_v.at[r % 2]
            gathers = []
            off = 0
            while off < n:
                c = min(128, n - off)
                gathers.append(
                    pltpu.async_copy(
                        tab_hbm.at[idx_v.at[pl.ds(start + off, c)]],
                        buf.at[pl.ds(off, c)],
                        gsem,
                    )
                )
                off += c
            if r >= 2:
                writes[r - 2].wait()
            for g in gathers:
                g.wait()
            writes.append(
                pltpu.async_copy(
                    buf.at[pl.ds(0, n)], out_hbm.at[b, pl.ds(start, n)], wsem
                )
            )
        for w in writes[-2:]:
            w.wait()

    return gather_kernel(idx_flat, table)


def _tc_body(aft_ref, ptt_ref, e_ref, wa_ref, wpt_ref, we_ref, b1_ref, g_ref, bt_ref, out_ref):
    cdims = (((0,), (0,)), ((), ()))
    vs = []
    for j in range(8):
        h = lax.dot_general(aft_ref[j], wa_ref[...], cdims,
                            preferred_element_type=jnp.float32)
        h = h + jnp.dot(e_ref[j], we_ref[...], preferred_element_type=jnp.float32)
        h = h + lax.dot_general(ptt_ref[j], wpt_ref[...], cdims,
                                preferred_element_type=jnp.float32)
        h = h + b1_ref[...]
        mu = jnp.mean(h, axis=1, keepdims=True)
        var = jnp.mean((h - mu) ** 2, axis=1, keepdims=True)
        hn = (h - mu) * lax.rsqrt(var + 1e-5)
        hn = hn * g_ref[...] + bt_ref[...]
        vs.append(0.5 * hn * (1.0 + lax.erf(hn * 0.7071067811865476)))
    out_ref[...] = jnp.stack(vs, axis=1)


def kernel(af, pi, pt, si, phoneme_table, speaker_table, W1, b1, gamma, beta):
    idx = pi.astype(jnp.int32) * 2 + si.astype(jnp.int32)
    idx_flat = jnp.pad(idx, ((0, 0), (0, T_PAD - T))).reshape(-1)

    comb = jnp.concatenate(
        [
            jnp.repeat(phoneme_table, 2, axis=0),
            jnp.tile(speaker_table, (128, 1)),
            jnp.zeros((256, E_DIM - PH_DIM - SP_DIM), jnp.float32),
        ],
        axis=1,
    )

    e = _sc_gather(idx_flat, comb)

    wa = W1[:FEAT_DIM]
    wpt = W1[FEAT_DIM:FEAT_DIM + 1]
    we = jnp.concatenate(
        [W1[FEAT_DIM + 1:], jnp.zeros((E_DIM - PH_DIM - SP_DIM, D_MODEL), jnp.float32)]
    )

    out_l = pl.pallas_call(
        _tc_body,
        grid=(B // 8,),
        in_specs=[
            pl.BlockSpec((8, FEAT_DIM, T), lambda i: (i, 0, 0)),
            pl.BlockSpec((8, 1, T), lambda i: (i, 0, 0)),
            pl.BlockSpec((8, T, E_DIM), lambda i: (i, 0, 0)),
            pl.BlockSpec((FEAT_DIM, D_MODEL), lambda i: (0, 0)),
            pl.BlockSpec((1, D_MODEL), lambda i: (0, 0)),
            pl.BlockSpec((E_DIM, D_MODEL), lambda i: (0, 0)),
            pl.BlockSpec((1, D_MODEL), lambda i: (0, 0)),
            pl.BlockSpec((1, D_MODEL), lambda i: (0, 0)),
            pl.BlockSpec((1, D_MODEL), lambda i: (0, 0)),
        ],
        out_specs=pl.BlockSpec((T, 8, D_MODEL), lambda i: (0, i, 0)),
        out_shape=jax.ShapeDtypeStruct((T, B, D_MODEL), jnp.float32),
        compiler_params=pltpu.CompilerParams(vmem_limit_bytes=100 * 1024 * 1024),
    )(af.transpose(0, 2, 1), pt.transpose(0, 2, 1), e, wa, wpt, we,
      b1.reshape(1, D_MODEL), gamma.reshape(1, D_MODEL), beta.reshape(1, D_MODEL))

    return out_l.transpose(1, 0, 2)
